# Initial kernel scaffold; baseline (speedup 1.0000x reference)
import jax
import jax.numpy as jnp
from jax.experimental import pallas as pl


def kernel(n, m, logits):
    raise NotImplementedError("write your pallas kernel here")


# speedup vs baseline: 1.0406x; 1.0406x over previous
import functools

import jax
import jax.numpy as jnp
from jax import lax
from jax.experimental import pallas as pl
from jax.experimental.pallas import tpu as pltpu
from jax.experimental.pallas import tpu_sc as plsc

_NC = 2
_NS = 16
_NW = _NC * _NS
_LANES = 16
_IDX_CHUNK = 128


def _lse_body(x_ref, o_ref):
    x = x_ref[...]
    mx = jnp.max(x)
    o_ref[0] = mx + jnp.log(jnp.sum(jnp.exp(x - mx)))


def _sub_body(g_ref, l_ref, o_ref):
    o_ref[...] = g_ref[...] - l_ref[0]


def _gather_body(cols, b_per_w, n_hbm, m_hbm, tab_hbm, out_hbm,
                 n_v, m_v, idx_v, val_v, sem):
    n_chunks = b_per_w // _IDX_CHUNK
    wid = lax.axis_index("s") * _NC + lax.axis_index("c")
    base = wid * b_per_w
    pltpu.sync_copy(n_hbm.at[pl.ds(base, b_per_w)], n_v)
    pltpu.sync_copy(m_hbm.at[pl.ds(base, b_per_w)], m_v)
    for j in range(n_chunks):
        for k in range(_IDX_CHUNK // _LANES):
            src = pl.ds(j * _IDX_CHUNK + k * _LANES, _LANES)
            idx_v[j, pl.ds(k * _LANES, _LANES)] = n_v[src] * cols + m_v[src]
    descs = [
        pltpu.async_copy(tab_hbm.at[idx_v.at[j]],
                         val_v.at[pl.ds(j * _IDX_CHUNK, _IDX_CHUNK)], sem)
        for j in range(n_chunks)
    ]
    for d in descs:
        d.wait()
    pltpu.sync_copy(val_v, out_hbm.at[pl.ds(base, b_per_w)])


def kernel(n, m, logits):
    rows, cols = logits.shape
    batch = n.shape[0]
    assert batch % (_NW * _IDX_CHUNK) == 0
    b_per_w = batch // _NW

    lse = pl.pallas_call(
        _lse_body,
        out_shape=jax.ShapeDtypeStruct((1,), jnp.float32),
        out_specs=pl.BlockSpec(memory_space=pltpu.SMEM),
    )(logits)

    gather = pl.kernel(
        functools.partial(_gather_body, cols, b_per_w),
        out_type=jax.ShapeDtypeStruct((batch,), jnp.float32),
        mesh=plsc.VectorSubcoreMesh(core_axis_name="c", subcore_axis_name="s"),
        scratch_types=[
            pltpu.VMEM((b_per_w,), jnp.int32),
            pltpu.VMEM((b_per_w,), jnp.int32),
            pltpu.VMEM((b_per_w // _IDX_CHUNK, _IDX_CHUNK), jnp.int32),
            pltpu.VMEM((b_per_w,), jnp.float32),
            pltpu.SemaphoreType.DMA,
        ],
    )
    g = gather(n.astype(jnp.int32), m.astype(jnp.int32), logits.reshape(-1))

    out = pl.pallas_call(
        _sub_body,
        out_shape=jax.ShapeDtypeStruct((batch // _IDX_CHUNK, _IDX_CHUNK),
                                       jnp.float32),
        in_specs=[
            pl.BlockSpec(memory_space=pltpu.VMEM),
            pl.BlockSpec(memory_space=pltpu.SMEM),
        ],
    )(g.reshape(batch // _IDX_CHUNK, _IDX_CHUNK), lse)
    return out.reshape(batch)
